# baseline (speedup 1.0000x reference)
import jax
import jax.numpy as jnp
from jax.experimental import pallas as pl
from jax.experimental.pallas import tpu as pltpu

_S = 64
_ROW = 256
_KW = 512


def _pqmf_mm_kernel(x_ref, hp_ref, hn_ref, wa_ref, wb_ref, wc_ref, o_ref):
    i = pl.program_id(1)
    n_t = pl.num_programs(1)
    X = x_ref[...]
    prev = jnp.where(i == 0, 0.0, hp_ref[7:8, :])
    nxt = jnp.where(i == n_t - 1, 0.0, hn_ref[0:1, :])
    A = jnp.concatenate([prev, X[:-1]], axis=0)
    C = jnp.concatenate([X[1:], nxt], axis=0)
    R = (jnp.dot(X, wb_ref[...], preferred_element_type=jnp.float32)
         + jnp.dot(A, wa_ref[...], preferred_element_type=jnp.float32)
         + jnp.dot(C, wc_ref[...], preferred_element_type=jnp.float32))
    for k in range(o_ref.shape[0]):
        o_ref[k] = R[:, k * _S:(k + 1) * _S]


def kernel(x, H):
    B, C, T = x.shape
    Nb, taps1 = H.shape
    Tq = T // Nb
    pad = (taps1 - 1) // 2

    x2 = x[:, 0, :]
    if T % _ROW:
        x2 = jnp.pad(x2, ((0, 0), (0, _ROW - T % _ROW)))
    G = x2.shape[1] // _ROW
    x3 = x2.reshape(B, G, _ROW)

    Gt = G
    for cand in (512, 256, 128, 64, 32, 16, 8):
        if G % cand == 0:
            Gt = cand
            break
    n_t = G // Gt

    c = jnp.arange(_KW)[:, None]
    s = jnp.arange(_S)[None, :]
    d = c - 4 * s
    valid = (d >= 0) & (d < taps1)
    Hg = H.astype(jnp.float32)[:, jnp.clip(d, 0, taps1 - 1)]
    Wfull = (jnp.where(valid[None], Hg, 0.0)
                .transpose(1, 0, 2)
                .reshape(_KW, Nb * _S))
    m = jnp.arange(_ROW)[:, None]
    Wa = jnp.where(m - (_ROW - pad) >= 0,
                   Wfull[jnp.clip(m[:, 0] - (_ROW - pad), 0, _KW - 1)], 0.0)
    Wb = Wfull[pad:pad + _ROW]
    Wc = jnp.where(m + _ROW + pad < _KW,
                   Wfull[jnp.clip(m[:, 0] + _ROW + pad, 0, _KW - 1)], 0.0)

    flops = 2 * B * G * 3 * _ROW * (Nb * _S)
    bytes_accessed = 4 * B * G * _ROW + 4 * B * Nb * G * _S

    gb = Gt // 8
    out = pl.pallas_call(
        _pqmf_mm_kernel,
        out_shape=jax.ShapeDtypeStruct((B, Nb, G, _S), jnp.float32),
        grid=(B, n_t),
        in_specs=[
            pl.BlockSpec((None, Gt, _ROW), lambda b, i: (b, i, 0)),
            pl.BlockSpec((None, 8, _ROW),
                         lambda b, i: (b, jnp.maximum(i * gb - 1, 0), 0)),
            pl.BlockSpec((None, 8, _ROW),
                         lambda b, i: (b, jnp.minimum((i + 1) * gb,
                                                      G // 8 - 1), 0)),
            pl.BlockSpec((_ROW, Nb * _S), lambda b, i: (0, 0)),
            pl.BlockSpec((_ROW, Nb * _S), lambda b, i: (0, 0)),
            pl.BlockSpec((_ROW, Nb * _S), lambda b, i: (0, 0)),
        ],
        out_specs=pl.BlockSpec((None, Nb, Gt, _S), lambda b, i: (b, 0, i, 0)),
        compiler_params=pltpu.CompilerParams(
            dimension_semantics=("parallel", "parallel")),
        cost_estimate=pl.CostEstimate(flops=int(flops), transcendentals=0,
                                      bytes_accessed=int(bytes_accessed)),
    )(x3, x3, x3, Wa, Wb, Wc)

    return out.reshape(B, Nb, G * _S)[:, :, :Tq]


# speedup vs baseline: 1.3672x; 1.3672x over previous
import jax
import jax.numpy as jnp
from jax.experimental import pallas as pl
from jax.experimental.pallas import tpu as pltpu

_S = 64
_ROW = 256
_KW = 512


def _pqmf_mm_kernel(x_ref, hp_ref, hn_ref, wa_ref, wb_ref, wc_ref, o_ref):
    i = pl.program_id(1)
    n_t = pl.num_programs(1)
    X = x_ref[...]
    prev = jnp.where(i == 0, 0.0, hp_ref[7:8, :])
    nxt = jnp.where(i == n_t - 1, 0.0, hn_ref[0:1, :])
    A = jnp.concatenate([prev, X[:-1]], axis=0)
    C = jnp.concatenate([X[1:], nxt], axis=0)
    R = (jnp.dot(X, wb_ref[...], preferred_element_type=jnp.float32)
         + jnp.dot(A, wa_ref[...], preferred_element_type=jnp.float32)
         + jnp.dot(C, wc_ref[...], preferred_element_type=jnp.float32))
    for k in range(o_ref.shape[0]):
        o_ref[k] = R[:, k * _S:(k + 1) * _S]


def kernel(x, H):
    B, C, T = x.shape
    Nb, taps1 = H.shape
    Tq = T // Nb
    pad = (taps1 - 1) // 2

    x2 = x[:, 0, :]
    if T % _ROW:
        x2 = jnp.pad(x2, ((0, 0), (0, _ROW - T % _ROW)))
    G = x2.shape[1] // _ROW
    x3 = x2.reshape(B, G, _ROW)

    Gt = G
    for cand in (512, 256, 128, 64, 32, 16, 8):
        if G % cand == 0:
            Gt = cand
            break
    n_t = G // Gt

    cc = jnp.arange(_KW)[:, None, None]
    ss = jnp.arange(_S)[None, :, None]
    dd = jnp.arange(taps1)[None, None, :]
    oh = (cc - 4 * ss == dd).astype(jnp.float32)
    Wfull = jnp.einsum('csd,kd->cks', oh, H.astype(jnp.float32),
                       precision=jax.lax.Precision.HIGHEST
                       ).reshape(_KW, Nb * _S)
    Wa = jnp.concatenate(
        [jnp.zeros((_ROW - pad, Nb * _S), jnp.float32), Wfull[:pad]], axis=0)
    Wb = Wfull[pad:pad + _ROW]
    Wc = jnp.concatenate(
        [Wfull[_ROW + pad:], jnp.zeros((pad, Nb * _S), jnp.float32)], axis=0)

    flops = 2 * B * G * 3 * _ROW * (Nb * _S)
    bytes_accessed = 4 * B * G * _ROW + 4 * B * Nb * G * _S

    gb = Gt // 8
    out = pl.pallas_call(
        _pqmf_mm_kernel,
        out_shape=jax.ShapeDtypeStruct((B, Nb, G, _S), jnp.float32),
        grid=(B, n_t),
        in_specs=[
            pl.BlockSpec((None, Gt, _ROW), lambda b, i: (b, i, 0)),
            pl.BlockSpec((None, 8, _ROW),
                         lambda b, i: (b, jnp.maximum(i * gb - 1, 0), 0)),
            pl.BlockSpec((None, 8, _ROW),
                         lambda b, i: (b, jnp.minimum((i + 1) * gb,
                                                      G // 8 - 1), 0)),
            pl.BlockSpec((_ROW, Nb * _S), lambda b, i: (0, 0)),
            pl.BlockSpec((_ROW, Nb * _S), lambda b, i: (0, 0)),
            pl.BlockSpec((_ROW, Nb * _S), lambda b, i: (0, 0)),
        ],
        out_specs=pl.BlockSpec((None, Nb, Gt, _S), lambda b, i: (b, 0, i, 0)),
        compiler_params=pltpu.CompilerParams(
            dimension_semantics=("parallel", "parallel")),
        cost_estimate=pl.CostEstimate(flops=int(flops), transcendentals=0,
                                      bytes_accessed=int(bytes_accessed)),
    )(x3, x3, x3, Wa, Wb, Wc)

    return out.reshape(B, Nb, G * _S)[:, :, :Tq]


# speedup vs baseline: 1.4085x; 1.0302x over previous
import jax
import jax.numpy as jnp
from jax.experimental import pallas as pl
from jax.experimental.pallas import tpu as pltpu

_S = 64
_ROW = 256
_KW = 512


def _pqmf_mm_kernel(x_ref, hp_ref, hn_ref, wa_ref, wb_ref, wc_ref, o_ref):
    i = pl.program_id(1)
    n_t = pl.num_programs(1)
    X = x_ref[...]
    prev = jnp.where(i == 0, 0.0, hp_ref[7:8, :])
    nxt = jnp.where(i == n_t - 1, 0.0, hn_ref[0:1, :])
    A = jnp.concatenate([prev, X[:-1]], axis=0)
    C = jnp.concatenate([X[1:], nxt], axis=0)
    R = (jnp.dot(X, wb_ref[...], preferred_element_type=jnp.float32)
         + jnp.dot(A, wa_ref[...], preferred_element_type=jnp.float32)
         + jnp.dot(C, wc_ref[...], preferred_element_type=jnp.float32))
    for k in range(o_ref.shape[0]):
        o_ref[k] = R[:, k * _S:(k + 1) * _S]


def kernel(x, H):
    B, C, T = x.shape
    Nb, taps1 = H.shape
    Tq = T // Nb
    pad = (taps1 - 1) // 2

    x2 = x[:, 0, :]
    if T % _ROW:
        x2 = jnp.pad(x2, ((0, 0), (0, _ROW - T % _ROW)))
    G = x2.shape[1] // _ROW
    x3 = x2.reshape(B, G, _ROW)

    Gt = G
    for cand in (512, 256, 128, 64, 32, 16, 8):
        if G % cand == 0:
            Gt = cand
            break
    n_t = G // Gt

    u = jnp.pad(H.astype(jnp.float32), ((0, 0), (0, _KW + 4 - taps1)))
    flat = jnp.tile(u, (1, _S))[:, :_S * _KW]
    Wfull = (flat.reshape(Nb, _S, _KW)
                 .transpose(2, 0, 1)
                 .reshape(_KW, Nb * _S))
    Wa = jnp.concatenate(
        [jnp.zeros((_ROW - pad, Nb * _S), jnp.float32), Wfull[:pad]], axis=0)
    Wb = Wfull[pad:pad + _ROW]
    Wc = jnp.concatenate(
        [Wfull[_ROW + pad:], jnp.zeros((pad, Nb * _S), jnp.float32)], axis=0)

    flops = 2 * B * G * 3 * _ROW * (Nb * _S)
    bytes_accessed = 4 * B * G * _ROW + 4 * B * Nb * G * _S

    gb = Gt // 8
    out = pl.pallas_call(
        _pqmf_mm_kernel,
        out_shape=jax.ShapeDtypeStruct((B, Nb, G, _S), jnp.float32),
        grid=(B, n_t),
        in_specs=[
            pl.BlockSpec((None, Gt, _ROW), lambda b, i: (b, i, 0)),
            pl.BlockSpec((None, 8, _ROW),
                         lambda b, i: (b, jnp.maximum(i * gb - 1, 0), 0)),
            pl.BlockSpec((None, 8, _ROW),
                         lambda b, i: (b, jnp.minimum((i + 1) * gb,
                                                      G // 8 - 1), 0)),
            pl.BlockSpec((_ROW, Nb * _S), lambda b, i: (0, 0)),
            pl.BlockSpec((_ROW, Nb * _S), lambda b, i: (0, 0)),
            pl.BlockSpec((_ROW, Nb * _S), lambda b, i: (0, 0)),
        ],
        out_specs=pl.BlockSpec((None, Nb, Gt, _S), lambda b, i: (b, 0, i, 0)),
        compiler_params=pltpu.CompilerParams(
            dimension_semantics=("parallel", "parallel")),
        cost_estimate=pl.CostEstimate(flops=int(flops), transcendentals=0,
                                      bytes_accessed=int(bytes_accessed)),
    )(x3, x3, x3, Wa, Wb, Wc)

    return out.reshape(B, Nb, G * _S)[:, :, :Tq]


# speedup vs baseline: 2.9677x; 2.1070x over previous
import jax
import jax.numpy as jnp
from jax.experimental import pallas as pl
from jax.experimental.pallas import tpu as pltpu

_S = 64
_ROW = 256
_FR = 128
_KW = 512


def _pqmf_mm_kernel(xf_ref, hp_ref, hn_ref, wa_ref, wb_ref, wc_ref, o_ref):
    i = pl.program_id(1)
    n_t = pl.num_programs(1)
    F = xf_ref[...]
    prev2 = jnp.where(i == 0, 0.0, hp_ref[6:8, :])
    next2 = jnp.where(i == n_t - 1, 0.0, hn_ref[0:2, :])
    fall = jnp.concatenate([prev2, F, next2], axis=0)
    xfull = fall.reshape(fall.shape[0] // 2, 2 * _FR)
    gt = xfull.shape[0] - 2
    A = xfull[0:gt]
    X = xfull[1:gt + 1]
    C = xfull[2:gt + 2]
    R = (jnp.dot(X, wb_ref[...], preferred_element_type=jnp.float32)
         + jnp.dot(A, wa_ref[...], preferred_element_type=jnp.float32)
         + jnp.dot(C, wc_ref[...], preferred_element_type=jnp.float32))
    for k in range(o_ref.shape[0]):
        o_ref[k] = R[:, k * _S:(k + 1) * _S]


def kernel(x, H):
    B, Cc, T = x.shape
    Nb, taps1 = H.shape
    Tq = T // Nb
    pad = (taps1 - 1) // 2

    x2 = x.reshape(B, T)
    if T % _ROW:
        x2 = jnp.pad(x2, ((0, 0), (0, _ROW - T % _ROW)))
    G = x2.shape[1] // _ROW
    xf = x2.reshape(B, 2 * G, _FR)

    Gt = G
    for cand in (512, 256, 128, 64, 32, 16, 8):
        if G % cand == 0:
            Gt = cand
            break
    n_t = G // Gt

    u = jnp.pad(H.astype(jnp.float32), ((0, 0), (0, _KW + 4 - taps1)))
    flat = jnp.tile(u, (1, _S))[:, :_S * _KW]
    Wfull = (flat.reshape(Nb, _S, _KW)
                 .transpose(2, 0, 1)
                 .reshape(_KW, Nb * _S))
    Wa = jnp.concatenate(
        [jnp.zeros((_ROW - pad, Nb * _S), jnp.float32), Wfull[:pad]], axis=0)
    Wb = Wfull[pad:pad + _ROW]
    Wc = jnp.concatenate(
        [Wfull[_ROW + pad:], jnp.zeros((pad, Nb * _S), jnp.float32)], axis=0)

    flops = 2 * B * G * 3 * _ROW * (Nb * _S)
    bytes_accessed = 4 * B * G * _ROW + 4 * B * Nb * G * _S

    fb = 2 * Gt // 8
    out = pl.pallas_call(
        _pqmf_mm_kernel,
        out_shape=jax.ShapeDtypeStruct((B, Nb, G, _S), jnp.float32),
        grid=(B, n_t),
        in_specs=[
            pl.BlockSpec((None, 2 * Gt, _FR), lambda b, i: (b, i, 0)),
            pl.BlockSpec((None, 8, _FR),
                         lambda b, i: (b, jnp.maximum(i * fb - 1, 0), 0)),
            pl.BlockSpec((None, 8, _FR),
                         lambda b, i: (b, jnp.minimum((i + 1) * fb,
                                                      2 * G // 8 - 1), 0)),
            pl.BlockSpec((_ROW, Nb * _S), lambda b, i: (0, 0)),
            pl.BlockSpec((_ROW, Nb * _S), lambda b, i: (0, 0)),
            pl.BlockSpec((_ROW, Nb * _S), lambda b, i: (0, 0)),
        ],
        out_specs=pl.BlockSpec((None, Nb, Gt, _S), lambda b, i: (b, 0, i, 0)),
        compiler_params=pltpu.CompilerParams(
            dimension_semantics=("parallel", "parallel")),
        cost_estimate=pl.CostEstimate(flops=int(flops), transcendentals=0,
                                      bytes_accessed=int(bytes_accessed)),
    )(xf, xf, xf, Wa, Wb, Wc)

    return out.reshape(B, Nb, G * _S)[:, :, :Tq]


# speedup vs baseline: 5.3393x; 1.7991x over previous
import jax
import jax.numpy as jnp
from jax.experimental import pallas as pl
from jax.experimental.pallas import tpu as pltpu

_S = 128
_CW = 512
_FR = 128
_KW = 576
_OFF = 225


def _pqmf_mm_kernel(xf_ref, hp_ref, hn_ref, v0_ref, v1_ref, o_ref):
    i = pl.program_id(1)
    n_t = pl.num_programs(1)
    F = xf_ref[...]
    prev6 = jnp.where(i == 0, 0.0, hp_ref[2:8, :])
    next2 = jnp.where(i == n_t - 1, 0.0, hn_ref[0:2, :])
    fall = jnp.concatenate([prev6, F, next2], axis=0)
    xf2 = fall.reshape(fall.shape[0] // 4, 4 * _FR)
    gt2 = xf2.shape[0] - 2
    Y0 = xf2[1:1 + gt2]
    Y1 = xf2[2:2 + gt2]
    R = (jnp.dot(Y0, v0_ref[...], preferred_element_type=jnp.float32)
         + jnp.dot(Y1, v1_ref[...], preferred_element_type=jnp.float32))
    o_ref[...] = R.reshape(4 * gt2, _FR)


def kernel(x, H):
    B, Cc, T = x.shape
    Nb, taps1 = H.shape
    Tq = T // Nb
    pad = (taps1 - 1) // 2

    x2 = x.reshape(B, T)
    if T % _CW:
        x2 = jnp.pad(x2, ((0, 0), (0, _CW - T % _CW)))
    Tp = x2.shape[1]
    G = Tp // 256
    xf = x2.reshape(B, 2 * G, _FR)

    Gt = G
    for cand in (512, 256, 128, 64, 32, 16, 8):
        if G % cand == 0:
            Gt = cand
            break
    n_t = G // Gt

    u = jnp.pad(H.astype(jnp.float32), ((0, 0), (0, _KW + 4 - taps1)))
    flat = jnp.tile(u, (1, _S))[:, :_S * _KW]
    W2 = (flat.reshape(Nb, _S, _KW)
              .transpose(2, 0, 1)
              .reshape(_KW, Nb * _S))
    V0 = jnp.concatenate(
        [jnp.zeros((_OFF, Nb * _S), jnp.float32), W2[:_CW - _OFF]], axis=0)
    V1 = jnp.concatenate(
        [W2[_CW - _OFF:], jnp.zeros((2 * _CW - _KW - _OFF, Nb * _S),
                                    jnp.float32)], axis=0)

    flops = 2 * B * (G // 2) * 2 * _CW * (Nb * _S)
    bytes_accessed = 4 * B * G * 256 + 4 * B * Nb * G * 64

    fb = 2 * Gt // 8
    out = pl.pallas_call(
        _pqmf_mm_kernel,
        out_shape=jax.ShapeDtypeStruct((B, 2 * G, _FR), jnp.float32),
        grid=(B, n_t),
        in_specs=[
            pl.BlockSpec((None, 2 * Gt, _FR), lambda b, i: (b, i, 0)),
            pl.BlockSpec((None, 8, _FR),
                         lambda b, i: (b, jnp.maximum(i * fb - 1, 0), 0)),
            pl.BlockSpec((None, 8, _FR),
                         lambda b, i: (b, jnp.minimum((i + 1) * fb,
                                                      2 * G // 8 - 1), 0)),
            pl.BlockSpec((_CW, Nb * _S), lambda b, i: (0, 0)),
            pl.BlockSpec((_CW, Nb * _S), lambda b, i: (0, 0)),
        ],
        out_specs=pl.BlockSpec((None, 2 * Gt, _FR), lambda b, i: (b, i, 0)),
        compiler_params=pltpu.CompilerParams(
            dimension_semantics=("parallel", "parallel")),
        cost_estimate=pl.CostEstimate(flops=int(flops), transcendentals=0,
                                      bytes_accessed=int(bytes_accessed)),
    )(xf, xf, xf, V0, V1)

    res = (out.reshape(B, G // 2, Nb, _S)
              .transpose(0, 2, 1, 3)
              .reshape(B, Nb, (G // 2) * _S))
    return res[:, :, :Tq]


# speedup vs baseline: 7.2452x; 1.3570x over previous
import jax
import jax.numpy as jnp
from jax.experimental import pallas as pl
from jax.experimental.pallas import tpu as pltpu

_S = 128
_CW = 512
_FR = 128
_KW = 576
_OFF = 225


def _pqmf_mm_kernel(xf_ref, hp_ref, hn_ref, v0_ref, v1_ref, o_ref):
    i = pl.program_id(1)
    n_t = pl.num_programs(1)
    F = xf_ref[...]
    prev6 = jnp.where(i == 0, 0.0, hp_ref[2:8, :])
    next2 = jnp.where(i == n_t - 1, 0.0, hn_ref[0:2, :])
    fall = jnp.concatenate([prev6, F, next2], axis=0)
    xf2 = fall.reshape(fall.shape[0] // 4, 4 * _FR)
    gt2 = xf2.shape[0] - 2
    Y0 = xf2[1:1 + gt2]
    Y1 = xf2[2:2 + gt2]
    R = (jnp.dot(Y0, v0_ref[...], preferred_element_type=jnp.float32)
         + jnp.dot(Y1, v1_ref[...], preferred_element_type=jnp.float32))
    o_ref[...] = R.reshape(4 * gt2, _FR)


def kernel(x, H):
    B, Cc, T = x.shape
    Nb, taps1 = H.shape
    Tq = T // Nb
    pad = (taps1 - 1) // 2

    x2 = x.reshape(B, T)
    if T % _CW:
        x2 = jnp.pad(x2, ((0, 0), (0, _CW - T % _CW)))
    Tp = x2.shape[1]
    G = Tp // 256
    xf = x2.reshape(B, 2 * G, _FR)

    Gt = G
    for cand in (1024, 512, 256, 128, 64, 32, 16, 8):
        if G % cand == 0:
            Gt = cand
            break
    n_t = G // Gt

    u = jnp.pad(H.astype(jnp.float32), ((0, 0), (0, _KW + 4 - taps1)))
    flat = jnp.tile(u, (1, _S))[:, :_S * _KW]
    W2 = (flat.reshape(Nb, _S, _KW)
              .transpose(2, 0, 1)
              .reshape(_KW, Nb * _S))
    V0 = jnp.concatenate(
        [jnp.zeros((_OFF, Nb * _S), jnp.float32), W2[:_CW - _OFF]], axis=0)
    V1 = jnp.concatenate(
        [W2[_CW - _OFF:], jnp.zeros((2 * _CW - _KW - _OFF, Nb * _S),
                                    jnp.float32)], axis=0)

    flops = 2 * B * (G // 2) * 2 * _CW * (Nb * _S)
    bytes_accessed = 4 * B * G * 256 + 4 * B * Nb * G * 64

    fb = 2 * Gt // 8
    out = pl.pallas_call(
        _pqmf_mm_kernel,
        out_shape=jax.ShapeDtypeStruct((B, 2 * G, _FR), jnp.float32),
        grid=(B, n_t),
        in_specs=[
            pl.BlockSpec((None, 2 * Gt, _FR), lambda b, i: (b, i, 0)),
            pl.BlockSpec((None, 8, _FR),
                         lambda b, i: (b, jnp.maximum(i * fb - 1, 0), 0)),
            pl.BlockSpec((None, 8, _FR),
                         lambda b, i: (b, jnp.minimum((i + 1) * fb,
                                                      2 * G // 8 - 1), 0)),
            pl.BlockSpec((_CW, Nb * _S), lambda b, i: (0, 0)),
            pl.BlockSpec((_CW, Nb * _S), lambda b, i: (0, 0)),
        ],
        out_specs=pl.BlockSpec((None, 2 * Gt, _FR), lambda b, i: (b, i, 0)),
        compiler_params=pltpu.CompilerParams(
            dimension_semantics=("parallel", "parallel")),
        cost_estimate=pl.CostEstimate(flops=int(flops), transcendentals=0,
                                      bytes_accessed=int(bytes_accessed)),
    )(xf, xf, xf, V0, V1)

    res = (out.reshape(B, G // 2, Nb, _S)
              .transpose(0, 2, 1, 3)
              .reshape(B, Nb, (G // 2) * _S))
    return res[:, :, :Tq]


# speedup vs baseline: 8.8484x; 1.2213x over previous
import jax
import jax.numpy as jnp
from jax.experimental import pallas as pl
from jax.experimental.pallas import tpu as pltpu

_S = 128
_CW = 512
_FR = 128
_KW = 576
_OFF = 225


def _pqmf_mm_kernel(xf_ref, hp_ref, hn_ref, v0_ref, v1_ref, o_ref):
    i = pl.program_id(1)
    n_t = pl.num_programs(1)
    for bl in range(xf_ref.shape[0]):
        F = xf_ref[bl]
        prev6 = jnp.where(i == 0, 0.0, hp_ref[bl, 2:8, :])
        next2 = jnp.where(i == n_t - 1, 0.0, hn_ref[bl, 0:2, :])
        fall = jnp.concatenate([prev6, F, next2], axis=0)
        xf2 = fall.reshape(fall.shape[0] // 4, 4 * _FR)
        gt2 = xf2.shape[0] - 2
        Y0 = xf2[1:1 + gt2]
        Y1 = xf2[2:2 + gt2]
        R = (jnp.dot(Y0, v0_ref[...], preferred_element_type=jnp.float32)
             + jnp.dot(Y1, v1_ref[...], preferred_element_type=jnp.float32))
        o_ref[bl] = R.reshape(4 * gt2, _FR)


def kernel(x, H):
    B, Cc, T = x.shape
    Nb, taps1 = H.shape
    Tq = T // Nb
    pad = (taps1 - 1) // 2

    x2 = x.reshape(B, T)
    if T % _CW:
        x2 = jnp.pad(x2, ((0, 0), (0, _CW - T % _CW)))
    Tp = x2.shape[1]
    G = Tp // 256
    xf = x2.reshape(B, 2 * G, _FR)

    Gt = G
    for cand in (1024, 512, 256, 128, 64, 32, 16, 8):
        if G % cand == 0:
            Gt = cand
            break
    n_t = G // Gt

    u = jnp.pad(H.astype(jnp.float32), ((0, 0), (0, _KW + 4 - taps1)))
    flat = jnp.tile(u, (1, _S))[:, :_S * _KW]
    W2 = (flat.reshape(Nb, _S, _KW)
              .transpose(2, 0, 1)
              .reshape(_KW, Nb * _S))
    V0 = jnp.concatenate(
        [jnp.zeros((_OFF, Nb * _S), jnp.float32), W2[:_CW - _OFF]], axis=0)
    V1 = jnp.concatenate(
        [W2[_CW - _OFF:], jnp.zeros((2 * _CW - _KW - _OFF, Nb * _S),
                                    jnp.float32)], axis=0)

    flops = 2 * B * (G // 2) * 2 * _CW * (Nb * _S)
    bytes_accessed = 4 * B * G * 256 + 4 * B * Nb * G * 64

    bb = 2 if B % 2 == 0 else 1
    fb = 2 * Gt // 8
    out = pl.pallas_call(
        _pqmf_mm_kernel,
        out_shape=jax.ShapeDtypeStruct((B, 2 * G, _FR), jnp.float32),
        grid=(B // bb, n_t),
        in_specs=[
            pl.BlockSpec((bb, 2 * Gt, _FR), lambda b, i: (b, i, 0)),
            pl.BlockSpec((bb, 8, _FR),
                         lambda b, i: (b, jnp.maximum(i * fb - 1, 0), 0)),
            pl.BlockSpec((bb, 8, _FR),
                         lambda b, i: (b, jnp.minimum((i + 1) * fb,
                                                      2 * G // 8 - 1), 0)),
            pl.BlockSpec((_CW, Nb * _S), lambda b, i: (0, 0)),
            pl.BlockSpec((_CW, Nb * _S), lambda b, i: (0, 0)),
        ],
        out_specs=pl.BlockSpec((bb, 2 * Gt, _FR), lambda b, i: (b, i, 0)),
        compiler_params=pltpu.CompilerParams(
            dimension_semantics=("parallel", "parallel")),
        cost_estimate=pl.CostEstimate(flops=int(flops), transcendentals=0,
                                      bytes_accessed=int(bytes_accessed)),
    )(xf, xf, xf, V0, V1)

    res = (out.reshape(B, G // 2, Nb, _S)
              .transpose(0, 2, 1, 3)
              .reshape(B, Nb, (G // 2) * _S))
    return res[:, :, :Tq]


# speedup vs baseline: 9.7426x; 1.1011x over previous
import jax
import jax.numpy as jnp
from jax.experimental import pallas as pl
from jax.experimental.pallas import tpu as pltpu

_S = 128
_CW = 512
_FR = 128
_KW = 576
_OFF = 225


def _pqmf_mm_kernel(xf_ref, hp_ref, hn_ref, v0_ref, v1_ref, o_ref):
    i = pl.program_id(1)
    n_t = pl.num_programs(1)
    for bl in range(xf_ref.shape[0]):
        F = xf_ref[bl]
        prev6 = jnp.where(i == 0, 0.0, hp_ref[bl, 2:8, :])
        next2 = jnp.where(i == n_t - 1, 0.0, hn_ref[bl, 0:2, :])
        fall = jnp.concatenate([prev6, F, next2], axis=0)
        xf2 = fall.reshape(fall.shape[0] // 4, 4 * _FR)
        gt2 = xf2.shape[0] - 2
        Y0 = xf2[1:1 + gt2]
        Y1 = xf2[2:2 + gt2]
        R = (jnp.dot(Y0, v0_ref[...], preferred_element_type=jnp.float32)
             + jnp.dot(Y1, v1_ref[...], preferred_element_type=jnp.float32))
        o_ref[bl] = R.reshape(4 * gt2, _FR)


def kernel(x, H):
    B, Cc, T = x.shape
    Nb, taps1 = H.shape
    Tq = T // Nb
    pad = (taps1 - 1) // 2

    x2 = x.reshape(B, T)
    if T % _CW:
        x2 = jnp.pad(x2, ((0, 0), (0, _CW - T % _CW)))
    Tp = x2.shape[1]
    G = Tp // 256
    xf = x2.reshape(B, 2 * G, _FR)

    Gt = G
    for cand in (1024, 512, 256, 128, 64, 32, 16, 8):
        if G % cand == 0:
            Gt = cand
            break
    n_t = G // Gt

    u = jnp.pad(H.astype(jnp.float32), ((0, 0), (0, _KW + 4 - taps1)))
    flat = jnp.tile(u, (1, _S))[:, :_S * _KW]
    W2 = (flat.reshape(Nb, _S, _KW)
              .transpose(2, 0, 1)
              .reshape(_KW, Nb * _S))
    V0 = jnp.concatenate(
        [jnp.zeros((_OFF, Nb * _S), jnp.float32), W2[:_CW - _OFF]], axis=0)
    V1 = jnp.concatenate(
        [W2[_CW - _OFF:], jnp.zeros((2 * _CW - _KW - _OFF, Nb * _S),
                                    jnp.float32)], axis=0)

    flops = 2 * B * (G // 2) * 2 * _CW * (Nb * _S)
    bytes_accessed = 4 * B * G * 256 + 4 * B * Nb * G * 64

    bb = 1
    for cand_b in (4, 2):
        if B % cand_b == 0:
            bb = cand_b
            break
    fb = 2 * Gt // 8
    out = pl.pallas_call(
        _pqmf_mm_kernel,
        out_shape=jax.ShapeDtypeStruct((B, 2 * G, _FR), jnp.float32),
        grid=(B // bb, n_t),
        in_specs=[
            pl.BlockSpec((bb, 2 * Gt, _FR), lambda b, i: (b, i, 0)),
            pl.BlockSpec((bb, 8, _FR),
                         lambda b, i: (b, jnp.maximum(i * fb - 1, 0), 0)),
            pl.BlockSpec((bb, 8, _FR),
                         lambda b, i: (b, jnp.minimum((i + 1) * fb,
                                                      2 * G // 8 - 1), 0)),
            pl.BlockSpec((_CW, Nb * _S), lambda b, i: (0, 0)),
            pl.BlockSpec((_CW, Nb * _S), lambda b, i: (0, 0)),
        ],
        out_specs=pl.BlockSpec((bb, 2 * Gt, _FR), lambda b, i: (b, i, 0)),
        compiler_params=pltpu.CompilerParams(
            dimension_semantics=("parallel", "parallel")),
        cost_estimate=pl.CostEstimate(flops=int(flops), transcendentals=0,
                                      bytes_accessed=int(bytes_accessed)),
    )(xf, xf, xf, V0, V1)

    res = (out.reshape(B, G // 2, Nb, _S)
              .transpose(0, 2, 1, 3)
              .reshape(B, Nb, (G // 2) * _S))
    return res[:, :, :Tq]


# speedup vs baseline: 9.7963x; 1.0055x over previous
import jax
import jax.numpy as jnp
from jax.experimental import pallas as pl
from jax.experimental.pallas import tpu as pltpu

_S = 128
_CW = 512
_FR = 128
_KW = 576
_OFF = 225


def _pqmf_mm_kernel(xf_ref, hp_ref, hn_ref, v0_ref, v1_ref, o_ref):
    i = pl.program_id(1)
    n_t = pl.num_programs(1)
    for bl in range(xf_ref.shape[0]):
        F = xf_ref[bl]
        prev6 = jnp.where(i == 0, 0.0, hp_ref[bl, 2:8, :])
        next2 = jnp.where(i == n_t - 1, 0.0, hn_ref[bl, 0:2, :])
        fall = jnp.concatenate([prev6, F, next2], axis=0)
        xf2 = fall.reshape(fall.shape[0] // 4, 4 * _FR)
        gt2 = xf2.shape[0] - 2
        Y0 = xf2[1:1 + gt2]
        Y1 = xf2[2:2 + gt2]
        R = (jnp.dot(Y0, v0_ref[...], preferred_element_type=jnp.float32)
             + jnp.dot(Y1, v1_ref[...], preferred_element_type=jnp.float32))
        o_ref[bl] = R.reshape(4 * gt2, _FR)


def kernel(x, H):
    B, Cc, T = x.shape
    Nb, taps1 = H.shape
    Tq = T // Nb
    pad = (taps1 - 1) // 2

    x2 = x.reshape(B, T)
    if T % _CW:
        x2 = jnp.pad(x2, ((0, 0), (0, _CW - T % _CW)))
    Tp = x2.shape[1]
    G = Tp // 256
    xf = x2.reshape(B, 2 * G, _FR)

    Gt = G
    for cand in (1024, 512, 256, 128, 64, 32, 16, 8):
        if G % cand == 0:
            Gt = cand
            break
    n_t = G // Gt

    u = jnp.pad(H.astype(jnp.float32), ((0, 0), (0, _KW + 4 - taps1)))
    flat = jnp.tile(u, (1, _S))[:, :_S * _KW]
    W2 = (flat.reshape(Nb, _S, _KW)
              .transpose(2, 0, 1)
              .reshape(_KW, Nb * _S))
    V0 = jnp.concatenate(
        [jnp.zeros((_OFF, Nb * _S), jnp.float32), W2[:_CW - _OFF]], axis=0)
    V1 = jnp.concatenate(
        [W2[_CW - _OFF:], jnp.zeros((2 * _CW - _KW - _OFF, Nb * _S),
                                    jnp.float32)], axis=0)

    flops = 2 * B * (G // 2) * 2 * _CW * (Nb * _S)
    bytes_accessed = 4 * B * G * 256 + 4 * B * Nb * G * 64

    bb = 1
    for cand_b in (8, 4, 2):
        if B % cand_b == 0:
            bb = cand_b
            break
    fb = 2 * Gt // 8
    out = pl.pallas_call(
        _pqmf_mm_kernel,
        out_shape=jax.ShapeDtypeStruct((B, 2 * G, _FR), jnp.float32),
        grid=(B // bb, n_t),
        in_specs=[
            pl.BlockSpec((bb, 2 * Gt, _FR), lambda b, i: (b, i, 0)),
            pl.BlockSpec((bb, 8, _FR),
                         lambda b, i: (b, jnp.maximum(i * fb - 1, 0), 0)),
            pl.BlockSpec((bb, 8, _FR),
                         lambda b, i: (b, jnp.minimum((i + 1) * fb,
                                                      2 * G // 8 - 1), 0)),
            pl.BlockSpec((_CW, Nb * _S), lambda b, i: (0, 0)),
            pl.BlockSpec((_CW, Nb * _S), lambda b, i: (0, 0)),
        ],
        out_specs=pl.BlockSpec((bb, 2 * Gt, _FR), lambda b, i: (b, i, 0)),
        compiler_params=pltpu.CompilerParams(
            dimension_semantics=("parallel", "parallel")),
        cost_estimate=pl.CostEstimate(flops=int(flops), transcendentals=0,
                                      bytes_accessed=int(bytes_accessed)),
    )(xf, xf, xf, V0, V1)

    res = (out.reshape(B, G // 2, Nb, _S)
              .transpose(0, 2, 1, 3)
              .reshape(B, Nb, (G // 2) * _S))
    return res[:, :, :Tq]


# speedup vs baseline: 10.1056x; 1.0316x over previous
import jax
import jax.numpy as jnp
from jax.experimental import pallas as pl
from jax.experimental.pallas import tpu as pltpu

_S = 128
_CW = 512
_FR = 128
_KW = 576
_OFF = 225


def _pqmf_mm_kernel(xf_ref, hp_ref, hn_ref, v0_ref, v1_ref, o_ref):
    i = pl.program_id(1)
    n_t = pl.num_programs(1)
    for bl in range(xf_ref.shape[0]):
        F = xf_ref[bl].astype(jnp.bfloat16)
        prev6 = jnp.where(i == 0, 0.0,
                          hp_ref[bl, 2:8, :]).astype(jnp.bfloat16)
        next2 = jnp.where(i == n_t - 1, 0.0,
                          hn_ref[bl, 0:2, :]).astype(jnp.bfloat16)
        fall = jnp.concatenate([prev6, F, next2], axis=0)
        xf2 = fall.reshape(fall.shape[0] // 4, 4 * _FR)
        gt2 = xf2.shape[0] - 2
        Y0 = xf2[1:1 + gt2]
        Y1 = xf2[2:2 + gt2]
        R = (jnp.dot(Y0, v0_ref[...], preferred_element_type=jnp.float32)
             + jnp.dot(Y1, v1_ref[...], preferred_element_type=jnp.float32))
        o_ref[bl] = R.reshape(4 * gt2, _FR)


def kernel(x, H):
    B, Cc, T = x.shape
    Nb, taps1 = H.shape
    Tq = T // Nb
    pad = (taps1 - 1) // 2

    x2 = x.reshape(B, T)
    if T % _CW:
        x2 = jnp.pad(x2, ((0, 0), (0, _CW - T % _CW)))
    Tp = x2.shape[1]
    G = Tp // 256
    xf = x2.reshape(B, 2 * G, _FR)

    Gt = G
    for cand in (1024, 512, 256, 128, 64, 32, 16, 8):
        if G % cand == 0:
            Gt = cand
            break
    n_t = G // Gt

    u = jnp.pad(H.astype(jnp.float32), ((0, 0), (0, _KW + 4 - taps1)))
    flat = jnp.tile(u, (1, _S))[:, :_S * _KW]
    W2 = (flat.reshape(Nb, _S, _KW)
              .transpose(2, 0, 1)
              .reshape(_KW, Nb * _S))
    V0 = jnp.concatenate(
        [jnp.zeros((_OFF, Nb * _S), jnp.float32), W2[:_CW - _OFF]],
        axis=0).astype(jnp.bfloat16)
    V1 = jnp.concatenate(
        [W2[_CW - _OFF:], jnp.zeros((2 * _CW - _KW - _OFF, Nb * _S),
                                    jnp.float32)], axis=0).astype(jnp.bfloat16)

    flops = 2 * B * (G // 2) * 2 * _CW * (Nb * _S)
    bytes_accessed = 4 * B * G * 256 + 4 * B * Nb * G * 64

    bb = 1
    for cand_b in (4, 2):
        if B % cand_b == 0:
            bb = cand_b
            break
    fb = 2 * Gt // 8
    out = pl.pallas_call(
        _pqmf_mm_kernel,
        out_shape=jax.ShapeDtypeStruct((B, 2 * G, _FR), jnp.float32),
        grid=(B // bb, n_t),
        in_specs=[
            pl.BlockSpec((bb, 2 * Gt, _FR), lambda b, i: (b, i, 0)),
            pl.BlockSpec((bb, 8, _FR),
                         lambda b, i: (b, jnp.maximum(i * fb - 1, 0), 0)),
            pl.BlockSpec((bb, 8, _FR),
                         lambda b, i: (b, jnp.minimum((i + 1) * fb,
                                                      2 * G // 8 - 1), 0)),
            pl.BlockSpec((_CW, Nb * _S), lambda b, i: (0, 0)),
            pl.BlockSpec((_CW, Nb * _S), lambda b, i: (0, 0)),
        ],
        out_specs=pl.BlockSpec((bb, 2 * Gt, _FR), lambda b, i: (b, i, 0)),
        compiler_params=pltpu.CompilerParams(
            dimension_semantics=("parallel", "parallel")),
        cost_estimate=pl.CostEstimate(flops=int(flops), transcendentals=0,
                                      bytes_accessed=int(bytes_accessed)),
    )(xf, xf, xf, V0, V1)

    res = (out.reshape(B, G // 2, Nb, _S)
              .transpose(0, 2, 1, 3)
              .reshape(B, Nb, (G // 2) * _S))
    return res[:, :, :Tq]
